# Initial kernel scaffold; baseline (speedup 1.0000x reference)
import jax
import jax.numpy as jnp
from jax.experimental import pallas as pl


def kernel(x, t, emb):
    raise NotImplementedError("write your pallas kernel here")


# speedup vs baseline: 3.0396x; 3.0396x over previous
import jax
import jax.numpy as jnp
from jax.experimental import pallas as pl

DIM = 1024
ROW_BLOCK = 1024


def _body(t_ref, x_ref, emb_ref, o_ref):
    tt = t_ref[0].reshape(ROW_BLOCK, 1)
    e0 = emb_ref[0, :][None, :]
    e1 = emb_ref[1, :][None, :]
    e2 = emb_ref[2, :][None, :]
    sel = jnp.where(tt == 0, e0, jnp.where(tt == 1, e1, e2))
    o_ref[...] = x_ref[...] + sel


def kernel(x, t, emb):
    b, s, d = x.shape
    rows = b * s
    nblk = rows // ROW_BLOCK
    x2 = x.reshape(rows, d)
    t3 = t.astype(jnp.int32).reshape(nblk, 1, ROW_BLOCK)
    out = pl.pallas_call(
        _body,
        grid=(nblk,),
        in_specs=[
            pl.BlockSpec((1, 1, ROW_BLOCK), lambda i: (i, 0, 0)),
            pl.BlockSpec((ROW_BLOCK, d), lambda i: (i, 0)),
            pl.BlockSpec((3, d), lambda i: (0, 0)),
        ],
        out_specs=pl.BlockSpec((ROW_BLOCK, d), lambda i: (i, 0)),
        out_shape=jax.ShapeDtypeStruct((rows, d), x.dtype),
    )(t3, x2, emb)
    return out.reshape(b, s, d)
